# baseline (speedup 1.0000x reference)
import functools

import jax
import jax.numpy as jnp
from jax import lax
from jax.experimental import pallas as pl
from jax.experimental.pallas import tpu as pltpu
from jax.experimental.pallas import tpu_sc as plsc

_OBJECT_NUM = 1024
_ATTR = 8
_ATTR_SHIFT = 3
_E_DIMS = 256
_BATCH = 4
_SEQ = 8192

_L = 16
_NW = 32
_SLICE = _SEQ // _NW

_mesh = plsc.VectorSubcoreMesh(core_axis_name="c", subcore_axis_name="s")


@functools.partial(
    pl.kernel,
    mesh=_mesh,
    out_type=jax.ShapeDtypeStruct((_BATCH * _SEQ, _E_DIMS), jnp.float32),
    scratch_types=[
        pltpu.VMEM((_SLICE // _ATTR, _E_DIMS), jnp.float32),
        pltpu.VMEM((_SLICE, _E_DIMS), jnp.float32),
        pltpu.SemaphoreType.DMA,
        pltpu.SemaphoreType.DMA,
    ],
)
def _sc_batch0(e_hbm, out_hbm, ebuf, buf, isem, osem):
    wid = lax.axis_index("s") * 2 + lax.axis_index("c")
    s0 = wid * _SLICE

    rows = _SLICE // _ATTR
    e0 = pl.multiple_of(lax.shift_right_logical(s0, _ATTR_SHIFT), rows)
    pltpu.async_copy(e_hbm.at[pl.ds(e0, rows)], ebuf, isem).wait()

    def expand(r, carry):
        for i in range(_E_DIMS // _L):
            v = ebuf[r, pl.ds(i * _L, _L)]
            for k in range(_ATTR):
                buf[r * _ATTR + k, pl.ds(i * _L, _L)] = v
        return carry

    lax.fori_loop(0, rows, expand, 0)

    pltpu.async_copy(buf, out_hbm.at[pl.ds(s0, _SLICE)], osem).wait()


def _tc_rest(e, flat):
    def body(e_ref, sc_ref, o_ref):
        del sc_ref
        x = e_ref[...]
        o_ref[...] = jnp.broadcast_to(
            x[:, None, :], (_OBJECT_NUM, _ATTR, _E_DIMS)
        ).reshape(_SEQ, _E_DIMS)

    return pl.pallas_call(
        body,
        grid=(_BATCH - 1,),
        in_specs=[
            pl.BlockSpec((_OBJECT_NUM, _E_DIMS), lambda j: (0, 0)),
            pl.BlockSpec(memory_space=pltpu.MemorySpace.HBM),
        ],
        out_specs=pl.BlockSpec((_SEQ, _E_DIMS), lambda j: (j + 1, 0)),
        out_shape=jax.ShapeDtypeStruct((_BATCH * _SEQ, _E_DIMS), jnp.float32),
        input_output_aliases={1: 0},
    )(e, flat)


def kernel(x, E_absolute_position):
    del x
    flat = _sc_batch0(E_absolute_position)
    flat = _tc_rest(E_absolute_position, flat)
    return flat.reshape(_BATCH, _SEQ, _E_DIMS)


# speedup vs baseline: 1.1069x; 1.1069x over previous
import functools

import jax
import jax.numpy as jnp
from jax import lax
from jax.experimental import pallas as pl
from jax.experimental.pallas import tpu as pltpu
from jax.experimental.pallas import tpu_sc as plsc

_OBJECT_NUM = 1024
_ATTR = 8
_ATTR_SHIFT = 3
_E_DIMS = 256
_BATCH = 4
_SEQ = 8192

_L = 16
_NW = 32
_SLICE = _SEQ // _NW
_CHUNK = 128
_NCHUNK = _SLICE // _CHUNK

_mesh = plsc.VectorSubcoreMesh(core_axis_name="c", subcore_axis_name="s")


@functools.partial(
    pl.kernel,
    mesh=_mesh,
    out_type=jax.ShapeDtypeStruct((_BATCH * _SEQ, _E_DIMS), jnp.float32),
    scratch_types=[
        pltpu.VMEM((_SLICE // _ATTR, _E_DIMS), jnp.float32),
        pltpu.VMEM((_SLICE, _E_DIMS), jnp.float32),
        pltpu.SemaphoreType.DMA,
        pltpu.SemaphoreType.DMA,
    ],
)
def _pos_broadcast(e_hbm, out_hbm, ebuf, buf, isem, osem):
    wid = lax.axis_index("s") * 2 + lax.axis_index("c")
    s0 = wid * _SLICE

    rows = _SLICE // _ATTR
    e0 = pl.multiple_of(lax.shift_right_logical(s0, _ATTR_SHIFT), rows)
    pltpu.async_copy(e_hbm.at[pl.ds(e0, rows)], ebuf, isem).wait()

    def expand(r, carry):
        for i in range(_E_DIMS // _L):
            v = ebuf[r, pl.ds(i * _L, _L)]
            for k in range(_ATTR):
                buf[r * _ATTR + k, pl.ds(i * _L, _L)] = v
        return carry

    lax.fori_loop(0, rows, expand, 0)

    outs = [
        pltpu.async_copy(buf, out_hbm.at[pl.ds(b * _SEQ + s0, _SLICE)], osem)
        for b in range(_BATCH)
    ]
    for o in outs:
        o.wait()


def kernel(x, E_absolute_position):
    del x
    flat = _pos_broadcast(E_absolute_position)
    return flat.reshape(_BATCH, _SEQ, _E_DIMS)
